# baseline (speedup 1.0000x reference)
import numpy as np
import jax
import jax.numpy as jnp
from jax import lax
from jax.experimental import pallas as pl
from jax.experimental.pallas import tpu as pltpu

B = 16384
N = 1000
R = 1024

_ROT_A = (13, 15, 26, 6)
_ROT_B = (17, 29, 16, 24)


def _rotl(x, r):
    return (x << np.uint32(r)) | (x >> np.uint32(32 - r))


def _threefry2x32(k0, k1, x0, x1):
    ks = (k0, k1, k0 ^ k1 ^ np.uint32(0x1BD11BDA))
    x0 = x0 + ks[0]
    x1 = x1 + ks[1]
    for i in range(5):
        for r in (_ROT_A if i % 2 == 0 else _ROT_B):
            x0 = x0 + x1
            x1 = _rotl(x1, r)
            x1 = x0 ^ x1
        x0 = x0 + ks[(i + 1) % 3]
        x1 = x1 + ks[(i + 2) % 3] + np.uint32(i + 1)
    return x0, x1


def _sample_kernel(wt_ref, seeds_ref, res_ref, hist_ref, loc_ref):

    seeds = seeds_ref[...].astype(jnp.uint32)
    zero = jnp.zeros_like(seeds)
    k0, k1 = _threefry2x32(zero, zero, zero, seeds)
    b0, b1 = _threefry2x32(k0, k1, zero, zero)
    bits = b0 ^ b1
    fbits = (bits >> np.uint32(9)) | np.uint32(0x3F800000)
    u = lax.bitcast_convert_type(fbits, jnp.float32) - jnp.float32(1.0)

    def scan_body(p, acc):
        acc = acc + wt_ref[p]
        loc_ref[p] = acc
        return acc
    tot = lax.fori_loop(0, 128, scan_body, jnp.zeros((8, R), jnp.float32),
                        unroll=8)

    offs_rows = [jnp.zeros((1, R), jnp.float32)]
    a = offs_rows[0]
    for g in range(7):
        a = a + tot[g:g + 1, :]
        offs_rows.append(a)
    offs = jnp.concatenate(offs_rows, axis=0)

    total = offs[7:8, :] + tot[7:8, :]
    r = total * (jnp.float32(1.0) - u)
    r8 = jnp.broadcast_to(r, (8, R))

    def cnt_body(i, accn):
        p = i * 8
        t = [(loc_ref[p + k] + offs < r8).astype(jnp.int32) for k in range(8)]
        s = ((t[0] + t[1]) + (t[2] + t[3])) + ((t[4] + t[5]) + (t[6] + t[7]))
        return accn + s
    cnt8 = lax.fori_loop(0, 16, cnt_body, jnp.zeros((8, R), jnp.int32))
    cnt = jnp.sum(cnt8, axis=0, keepdims=True)
    res_ref[...] = cnt

    iota = lax.broadcasted_iota(jnp.int32, (N, R), 0)
    hist_ref[...] = (iota == jnp.broadcast_to(cnt, (N, R))).astype(jnp.int32)


NCHUNK = 4
CB = B // NCHUNK


def _run_chunk(wchunk, schunk):
    wt3 = jnp.pad(wchunk, ((0, 0), (0, 24))).reshape(CB, 8, 128).transpose(2, 1, 0)
    seeds2 = schunk.reshape(1, CB)
    return pl.pallas_call(
        _sample_kernel,
        grid=(CB // R,),
        in_specs=[
            pl.BlockSpec((128, 8, R), lambda i: (0, 0, i)),
            pl.BlockSpec((1, R), lambda i: (0, i)),
        ],
        out_specs=[
            pl.BlockSpec((1, R), lambda i: (0, i)),
            pl.BlockSpec((N, R), lambda i: (0, i)),
        ],
        out_shape=[
            jax.ShapeDtypeStruct((1, CB), jnp.int32),
            jax.ShapeDtypeStruct((N, CB), jnp.int32),
        ],
        scratch_shapes=[pltpu.VMEM((128, 8, R), jnp.float32)],
    )(wt3, seeds2)


def kernel(weights, hist, seeds):
    del hist
    parts = [_run_chunk(weights[k * CB:(k + 1) * CB],
                        seeds[k * CB:(k + 1) * CB]) for k in range(NCHUNK)]
    res = jnp.concatenate([p[0] for p in parts], axis=1).reshape(B)
    hist_new = jnp.concatenate([p[1] for p in parts], axis=1)
    return res, hist_new.T


# speedup vs baseline: 3.7876x; 3.7876x over previous
import numpy as np
import jax
import jax.numpy as jnp
from jax import lax
from jax.experimental import pallas as pl
from jax.experimental.pallas import tpu as pltpu

B = 16384
N = 1000
R = 1024

_ROT_A = (13, 15, 26, 6)
_ROT_B = (17, 29, 16, 24)


def _rotl(x, r):
    return (x << np.uint32(r)) | (x >> np.uint32(32 - r))


def _threefry2x32(k0, k1, x0, x1):
    ks = (k0, k1, k0 ^ k1 ^ np.uint32(0x1BD11BDA))
    x0 = x0 + ks[0]
    x1 = x1 + ks[1]
    for i in range(5):
        for r in (_ROT_A if i % 2 == 0 else _ROT_B):
            x0 = x0 + x1
            x1 = _rotl(x1, r)
            x1 = x0 ^ x1
        x0 = x0 + ks[(i + 1) % 3]
        x1 = x1 + ks[(i + 2) % 3] + np.uint32(i + 1)
    return x0, x1


def _sample_kernel(wt_ref, seeds_ref, res_ref, hist_ref, loc_ref):

    seeds = seeds_ref[...].astype(jnp.uint32)
    zero = jnp.zeros_like(seeds)
    k0, k1 = _threefry2x32(zero, zero, zero, seeds)
    b0, b1 = _threefry2x32(k0, k1, zero, zero)
    bits = b0 ^ b1
    fbits = (bits >> np.uint32(9)) | np.uint32(0x3F800000)
    u = lax.bitcast_convert_type(fbits, jnp.float32) - jnp.float32(1.0)

    acc = jnp.zeros((8, R), jnp.float32)
    tot7 = None
    for p in range(128):
        wp = jnp.concatenate([wt_ref[p + 128 * g:p + 128 * g + 1, :]
                              for g in range(8)], axis=0)
        acc = acc + wp
        loc_ref[p] = acc
        if p == 103:
            tot7 = acc[7:8, :]
    tot = acc

    offs_rows = [jnp.zeros((1, R), jnp.float32)]
    a = offs_rows[0]
    for g in range(7):
        a = a + tot[g:g + 1, :]
        offs_rows.append(a)
    offs = jnp.concatenate(offs_rows, axis=0)

    total = offs[7:8, :] + tot7
    r = total * (jnp.float32(1.0) - u)
    r8 = jnp.broadcast_to(r, (8, R))

    m7 = (lax.broadcasted_iota(jnp.int32, (8, R), 0) < 7).astype(jnp.int32)

    def cnt_body(i, accn):
        p = i * 8
        t = [(loc_ref[p + k] + offs < r8).astype(jnp.int32) for k in range(8)]
        s = ((t[0] + t[1]) + (t[2] + t[3])) + ((t[4] + t[5]) + (t[6] + t[7]))
        return accn + s

    def cnt_body_m(i, accn):
        p = i * 8
        t = [(loc_ref[p + k] + offs < r8).astype(jnp.int32) for k in range(8)]
        s = ((t[0] + t[1]) + (t[2] + t[3])) + ((t[4] + t[5]) + (t[6] + t[7]))
        return accn + s * m7

    cnt8 = lax.fori_loop(0, 13, cnt_body, jnp.zeros((8, R), jnp.int32))
    cnt8 = lax.fori_loop(13, 16, cnt_body_m, cnt8)
    cnt = jnp.sum(cnt8, axis=0, keepdims=True)
    res_ref[...] = cnt

    iota = lax.broadcasted_iota(jnp.int32, (N, R), 0)
    hist_ref[...] = (iota == jnp.broadcast_to(cnt, (N, R))).astype(jnp.int32)


def kernel(weights, hist, seeds):
    del hist
    wt3 = weights.T
    seeds2 = seeds.reshape(1, B)
    res, hist_new = pl.pallas_call(
        _sample_kernel,
        grid=(B // R,),
        in_specs=[
            pl.BlockSpec((1024, R), lambda i: (0, i)),
            pl.BlockSpec((1, R), lambda i: (0, i)),
        ],
        out_specs=[
            pl.BlockSpec((1, R), lambda i: (0, i)),
            pl.BlockSpec((N, R), lambda i: (0, i)),
        ],
        out_shape=[
            jax.ShapeDtypeStruct((1, B), jnp.int32),
            jax.ShapeDtypeStruct((N, B), jnp.int32),
        ],
        scratch_shapes=[pltpu.VMEM((128, 8, R), jnp.float32)],
    )(wt3, seeds2)
    return res.reshape(B), hist_new.T


# speedup vs baseline: 3.8522x; 1.0171x over previous
import numpy as np
import jax
import jax.numpy as jnp
from jax import lax
from jax.experimental import pallas as pl
from jax.experimental.pallas import tpu as pltpu

B = 16384
N = 1000
R = 512

_ROT_A = (13, 15, 26, 6)
_ROT_B = (17, 29, 16, 24)


def _rotl(x, r):
    return (x << np.uint32(r)) | (x >> np.uint32(32 - r))


def _threefry2x32(k0, k1, x0, x1):
    ks = (k0, k1, k0 ^ k1 ^ np.uint32(0x1BD11BDA))
    x0 = x0 + ks[0]
    x1 = x1 + ks[1]
    for i in range(5):
        for r in (_ROT_A if i % 2 == 0 else _ROT_B):
            x0 = x0 + x1
            x1 = _rotl(x1, r)
            x1 = x0 ^ x1
        x0 = x0 + ks[(i + 1) % 3]
        x1 = x1 + ks[(i + 2) % 3] + np.uint32(i + 1)
    return x0, x1


def _sample_kernel(wt_ref, seeds_ref, res_ref, hist_ref, loc_ref):

    seeds = seeds_ref[...].astype(jnp.uint32)
    zero = jnp.zeros_like(seeds)
    k0, k1 = _threefry2x32(zero, zero, zero, seeds)
    b0, b1 = _threefry2x32(k0, k1, zero, zero)
    bits = b0 ^ b1
    fbits = (bits >> np.uint32(9)) | np.uint32(0x3F800000)
    u = lax.bitcast_convert_type(fbits, jnp.float32) - jnp.float32(1.0)

    acc = jnp.zeros((8, R), jnp.float32)
    tot7 = None
    for p in range(128):
        wp = jnp.concatenate([wt_ref[p + 128 * g:p + 128 * g + 1, :]
                              for g in range(8)], axis=0)
        acc = acc + wp
        loc_ref[p] = acc
        if p == 103:
            tot7 = acc[7:8, :]
    tot = acc

    offs_rows = [jnp.zeros((1, R), jnp.float32)]
    a = offs_rows[0]
    for g in range(7):
        a = a + tot[g:g + 1, :]
        offs_rows.append(a)
    offs = jnp.concatenate(offs_rows, axis=0)

    total = offs[7:8, :] + tot7
    r = total * (jnp.float32(1.0) - u)
    r8 = jnp.broadcast_to(r, (8, R))

    m7 = (lax.broadcasted_iota(jnp.int32, (8, R), 0) < 7).astype(jnp.int32)

    def cnt_body(i, accn):
        p = i * 8
        t = [(loc_ref[p + k] + offs < r8).astype(jnp.int32) for k in range(8)]
        s = ((t[0] + t[1]) + (t[2] + t[3])) + ((t[4] + t[5]) + (t[6] + t[7]))
        return accn + s

    def cnt_body_m(i, accn):
        p = i * 8
        t = [(loc_ref[p + k] + offs < r8).astype(jnp.int32) for k in range(8)]
        s = ((t[0] + t[1]) + (t[2] + t[3])) + ((t[4] + t[5]) + (t[6] + t[7]))
        return accn + s * m7

    cnt8 = lax.fori_loop(0, 13, cnt_body, jnp.zeros((8, R), jnp.int32))
    cnt8 = lax.fori_loop(13, 16, cnt_body_m, cnt8)
    cnt = jnp.sum(cnt8, axis=0, keepdims=True)
    res_ref[...] = cnt

    iota = lax.broadcasted_iota(jnp.int32, (N, R), 0)
    hist_ref[...] = (iota == jnp.broadcast_to(cnt, (N, R))).astype(jnp.int32)


def kernel(weights, hist, seeds):
    del hist
    wt3 = weights.T
    seeds2 = seeds.reshape(1, B)
    res, hist_new = pl.pallas_call(
        _sample_kernel,
        grid=(B // R,),
        in_specs=[
            pl.BlockSpec((1024, R), lambda i: (0, i)),
            pl.BlockSpec((1, R), lambda i: (0, i)),
        ],
        out_specs=[
            pl.BlockSpec((1, R), lambda i: (0, i)),
            pl.BlockSpec((N, R), lambda i: (0, i)),
        ],
        out_shape=[
            jax.ShapeDtypeStruct((1, B), jnp.int32),
            jax.ShapeDtypeStruct((N, B), jnp.int32),
        ],
        scratch_shapes=[pltpu.VMEM((128, 8, R), jnp.float32)],
    )(wt3, seeds2)
    return res.reshape(B), hist_new.T
